# Initial kernel scaffold; baseline (speedup 1.0000x reference)
import jax
import jax.numpy as jnp
from jax.experimental import pallas as pl


def kernel(n_layer, source_node_features, source_nodes_time_embedding, neighbor_embeddings, edge_time_embeddings, edge_features, mask, W1, b1, W2, b2):
    raise NotImplementedError("write your pallas kernel here")


# speedup vs baseline: 1.5842x; 1.5842x over previous
import functools

import jax
import jax.numpy as jnp
from jax.experimental import pallas as pl
from jax.experimental.pallas import tpu as pltpu

B, NB = 10000, 32
D, DT, DE = 128, 128, 16
TB = 200


def _body(ne_ref, ete_ref, ef_ref, src_ref, tim_ref,
          w1a_ref, w1b_ref, w1ct_ref, b1_ref,
          w2a_ref, w2b_ref, w2c_ref, b2_ref, out_ref):
    ne_sum = jnp.sum(ne_ref[...], axis=1)
    ete_sum = jnp.sum(ete_ref[...], axis=1)
    acc = (
        jnp.dot(ne_sum, w1a_ref[...], preferred_element_type=jnp.float32)
        + jnp.dot(ete_sum, w1b_ref[...], preferred_element_type=jnp.float32)
        + jnp.dot(ef_ref[...], w1ct_ref[...], preferred_element_type=jnp.float32)
    )
    h = jnp.maximum(acc + b1_ref[...] * float(NB), 0.0)
    out = (
        jnp.dot(h, w2a_ref[...], preferred_element_type=jnp.float32)
        + jnp.dot(src_ref[...], w2b_ref[...], preferred_element_type=jnp.float32)
        + jnp.dot(tim_ref[...], w2c_ref[...], preferred_element_type=jnp.float32)
        + b2_ref[...]
    )
    out_ref[...] = out


@functools.partial(jax.jit, static_argnums=())
def _run(ne, ete, ef_flat, src, tim, w1a, w1b, w1c_t, b1, w2a, w2b, w2c, b2):
    grid = (B // TB,)
    full = lambda i: (0, 0)
    return pl.pallas_call(
        _body,
        grid=grid,
        in_specs=[
            pl.BlockSpec((TB, NB, D), lambda i: (i, 0, 0)),
            pl.BlockSpec((TB, NB, DT), lambda i: (i, 0, 0)),
            pl.BlockSpec((TB, NB * DE), lambda i: (i, 0)),
            pl.BlockSpec((TB, D), lambda i: (i, 0)),
            pl.BlockSpec((TB, DT), lambda i: (i, 0)),
            pl.BlockSpec((D, D), full),
            pl.BlockSpec((DT, D), full),
            pl.BlockSpec((NB * DE, D), full),
            pl.BlockSpec((1, D), full),
            pl.BlockSpec((D, D), full),
            pl.BlockSpec((D, D), full),
            pl.BlockSpec((DT, D), full),
            pl.BlockSpec((1, D), full),
        ],
        out_specs=pl.BlockSpec((TB, D), lambda i: (i, 0)),
        out_shape=jax.ShapeDtypeStruct((B, D), jnp.float32),
    )(ne, ete, ef_flat, src, tim, w1a, w1b, w1c_t, b1, w2a, w2b, w2c, b2)


def kernel(n_layer, source_node_features, source_nodes_time_embedding,
           neighbor_embeddings, edge_time_embeddings, edge_features, mask,
           W1, b1, W2, b2):
    ef_flat = edge_features.reshape(B, NB * DE)
    tim = source_nodes_time_embedding.reshape(B, DT)
    w1a = W1[:D]
    w1b = W1[D:D + DT]
    w1c_t = jnp.tile(W1[D + DT:], (NB, 1))
    w2a = W2[:D]
    w2b = W2[D:2 * D]
    w2c = W2[2 * D:]
    return _run(neighbor_embeddings, edge_time_embeddings, ef_flat,
                source_node_features, tim, w1a, w1b, w1c_t,
                b1.reshape(1, D), w2a, w2b, w2c, b2.reshape(1, D))


# speedup vs baseline: 1.6019x; 1.0112x over previous
import functools

import jax
import jax.numpy as jnp
from jax.experimental import pallas as pl
from jax.experimental.pallas import tpu as pltpu

B, NB = 10000, 32
D, DT, DE = 128, 128, 16
TB = 400


def _body(ne_ref, ete_ref, ef_ref, src_ref, tim_ref,
          w1a_ref, w1b_ref, w1ct_ref, b1_ref,
          w2a_ref, w2b_ref, w2c_ref, b2_ref, out_ref):
    ne_sum = jnp.sum(ne_ref[...], axis=1)
    ete_sum = jnp.sum(ete_ref[...], axis=1)
    acc = (
        jnp.dot(ne_sum, w1a_ref[...], preferred_element_type=jnp.float32)
        + jnp.dot(ete_sum, w1b_ref[...], preferred_element_type=jnp.float32)
        + jnp.dot(ef_ref[...], w1ct_ref[...], preferred_element_type=jnp.float32)
    )
    h = jnp.maximum(acc + b1_ref[...] * float(NB), 0.0)
    out = (
        jnp.dot(h, w2a_ref[...], preferred_element_type=jnp.float32)
        + jnp.dot(src_ref[...], w2b_ref[...], preferred_element_type=jnp.float32)
        + jnp.dot(tim_ref[...], w2c_ref[...], preferred_element_type=jnp.float32)
        + b2_ref[...]
    )
    out_ref[...] = out


@functools.partial(jax.jit, static_argnums=())
def _run(ne, ete, ef_flat, src, tim, w1a, w1b, w1c_t, b1, w2a, w2b, w2c, b2):
    grid = (B // TB,)
    full = lambda i: (0, 0)
    return pl.pallas_call(
        _body,
        grid=grid,
        in_specs=[
            pl.BlockSpec((TB, NB, D), lambda i: (i, 0, 0)),
            pl.BlockSpec((TB, NB, DT), lambda i: (i, 0, 0)),
            pl.BlockSpec((TB, NB * DE), lambda i: (i, 0)),
            pl.BlockSpec((TB, D), lambda i: (i, 0)),
            pl.BlockSpec((TB, DT), lambda i: (i, 0)),
            pl.BlockSpec((D, D), full),
            pl.BlockSpec((DT, D), full),
            pl.BlockSpec((NB * DE, D), full),
            pl.BlockSpec((1, D), full),
            pl.BlockSpec((D, D), full),
            pl.BlockSpec((D, D), full),
            pl.BlockSpec((DT, D), full),
            pl.BlockSpec((1, D), full),
        ],
        out_specs=pl.BlockSpec((TB, D), lambda i: (i, 0)),
        out_shape=jax.ShapeDtypeStruct((B, D), jnp.float32),
    )(ne, ete, ef_flat, src, tim, w1a, w1b, w1c_t, b1, w2a, w2b, w2c, b2)


def kernel(n_layer, source_node_features, source_nodes_time_embedding,
           neighbor_embeddings, edge_time_embeddings, edge_features, mask,
           W1, b1, W2, b2):
    ef_flat = edge_features.reshape(B, NB * DE)
    tim = source_nodes_time_embedding.reshape(B, DT)
    w1a = W1[:D]
    w1b = W1[D:D + DT]
    w1c_t = jnp.tile(W1[D + DT:], (NB, 1))
    w2a = W2[:D]
    w2b = W2[D:2 * D]
    w2c = W2[2 * D:]
    return _run(neighbor_embeddings, edge_time_embeddings, ef_flat,
                source_node_features, tim, w1a, w1b, w1c_t,
                b1.reshape(1, D), w2a, w2b, w2c, b2.reshape(1, D))
